# baseline (speedup 1.0000x reference)
import functools

import jax
import jax.numpy as jnp
from jax import lax
from jax.experimental import pallas as pl
from jax.experimental.pallas import tpu as pltpu
from jax.experimental.pallas import tpu_sc as plsc

C, L1, D = 64, 2047, 256
NUM_BINS, HID = 32, 128
L = L1 + 1
N = C * L
TWO_D = 2 * D

NC, NS = 2, 16
NW = NC * NS
ROWS_PER_W = N // NW
BLK = 32
NCHUNK = ROWS_PER_W // BLK
RING = 4
LAG = 2


def _expr_table_body(bins_ref, w1_ref, b1_ref, w2_ref, b2_ref, out_ref):
    nb, d = out_ref.shape
    bins = bins_ref[...]
    vals = lax.broadcasted_iota(jnp.int32, (nb, 1), 0).astype(jnp.float32)
    h = jnp.maximum(vals * w1_ref[...] + b1_ref[...], 0.0)
    enc = lax.dot_general(h, w2_ref[...], (((1,), (0,)), ((), ())),
                          precision=lax.Precision.DEFAULT) + b2_ref[...]
    sim = lax.dot_general(enc, bins, (((1,), (1,)), ((), ())),
                          precision=lax.Precision.DEFAULT)
    col = lax.broadcasted_iota(jnp.int32, (nb, nb), 1)
    sim = jnp.where(col == 0, -1e30, sim)
    m = jnp.max(sim, axis=-1, keepdims=True)
    p = jnp.exp(sim - m)
    p = p / jnp.sum(p, axis=-1, keepdims=True)
    emb = lax.dot_general(p, bins, (((1,), (0,)), ((), ())),
                          precision=lax.Precision.DEFAULT)
    row = lax.broadcasted_iota(jnp.int32, (nb, d), 0)
    out_ref[...] = jnp.where(row == 0, bins[0:1, :], emb)


def _expr_table(bin_embeddings, w1, b1, w2, b2):
    return pl.pallas_call(
        _expr_table_body,
        out_shape=jax.ShapeDtypeStruct((NUM_BINS, D), jnp.float32),
    )(bin_embeddings, w1.reshape(1, HID), b1.reshape(1, HID),
      w2, b2.reshape(1, D))


def _sc_body(gidx_hbm, eidx_hbm, cidx_hbm, bidx_hbm,
             gene_t, expr_t, cond_t, batch_t, out_hbm,
             gidx_v, eidx_v, obuf0, obuf1, obuf2, obuf3,
             ci_v, bi_v, cbuf, bbuf, fbuf,
             sem_g0, sem_g1, sem_g2, sem_g3,
             sem_w0, sem_w1, sem_w2, sem_w3):
    obuf = (obuf0, obuf1, obuf2, obuf3)
    sem_g = (sem_g0, sem_g1, sem_g2, sem_g3)
    sem_w = (sem_w0, sem_w1, sem_w2, sem_w3)

    wid = lax.axis_index("s") * NC + lax.axis_index("c")
    base_w = pl.multiple_of(wid * ROWS_PER_W, ROWS_PER_W)
    chunk0 = pl.multiple_of(wid * NCHUNK, NCHUNK)

    pltpu.sync_copy(gidx_hbm.at[pl.ds(chunk0, NCHUNK)], gidx_v)
    pltpu.sync_copy(eidx_hbm.at[pl.ds(chunk0, NCHUNK)], eidx_v)

    def issue_gathers(i, p):
        pltpu.async_copy(gene_t.at[gidx_v.at[i]], obuf[p].at[:, pl.ds(0, D)],
                         sem_g[p])
        pltpu.async_copy(expr_t.at[eidx_v.at[i]], obuf[p].at[:, pl.ds(D, D)],
                         sem_g[p])

    def wait_gathers(i, p):
        pltpu.make_async_copy(gene_t.at[gidx_v.at[i]],
                              obuf[p].at[:, pl.ds(0, D)], sem_g[p]).wait()
        pltpu.make_async_copy(expr_t.at[eidx_v.at[i]],
                              obuf[p].at[:, pl.ds(D, D)], sem_g[p]).wait()

    def issue_write(i, p):
        base = pl.multiple_of(base_w + i * BLK, BLK)
        pltpu.async_copy(obuf[p], out_hbm.at[pl.ds(base, BLK)], sem_w[p])

    def wait_write(p):
        pltpu.make_async_copy(obuf[p], out_hbm.at[pl.ds(base_w, BLK)],
                              sem_w[p]).wait()

    for i in range(LAG):
        issue_gathers(i, i)

    def step(i, p):
        pb = (p + LAG) % RING

        @pl.when(i + LAG < NCHUNK)
        def _():
            @pl.when(i + LAG >= RING)
            def _():
                wait_write(pb)
            issue_gathers(i + LAG, pb)

        wait_gathers(i, p)
        issue_write(i, p)

    def group(j, carry):
        for u in range(RING):
            step(RING * j + u, u)
        return carry

    lax.fori_loop(0, NCHUNK // RING, group, 0)
    wait_write((NCHUNK - 2) % RING)
    wait_write((NCHUNK - 1) % RING)

    c0 = 2 * wid
    pltpu.sync_copy(cidx_hbm.at[pl.ds(pl.multiple_of(c0 * 4, 8), 8)], ci_v)
    pltpu.sync_copy(bidx_hbm.at[pl.ds(pl.multiple_of(wid * 8, 8), 8)], bi_v)
    c_dma = pltpu.async_copy(cond_t.at[ci_v], cbuf, sem_g0)
    b_dma = pltpu.async_copy(batch_t.at[bi_v], bbuf, sem_g1)
    c_dma.wait()
    b_dma.wait()
    for cell in range(2):
        row = (c0 + cell) * L
        for j in range(D // 16):
            p = j * 16
            fbuf[0, pl.ds(p, 16)] = cbuf[4 * cell + p // 64, pl.ds(p % 64, 16)]
        pltpu.sync_copy(fbuf, out_hbm.at[pl.ds(row, 1), pl.ds(0, D)])
        pltpu.sync_copy(bbuf.at[cell], out_hbm.at[row, pl.ds(D, D)])


@functools.partial(jax.jit, static_argnames=())
def _sc_assemble(gidx, eidx, cidx, bidx, gene_table, expr_table,
                 cond_table, batch_table):
    mesh = plsc.VectorSubcoreMesh(core_axis_name="c", subcore_axis_name="s")
    return pl.kernel(
        _sc_body,
        out_type=jax.ShapeDtypeStruct((N, TWO_D), jnp.float32),
        mesh=mesh,
        scratch_types=(
            [pltpu.VMEM((NCHUNK, BLK), jnp.int32)] * 2
            + [pltpu.VMEM((BLK, TWO_D), jnp.float32)] * 4
            + [
                pltpu.VMEM((8,), jnp.int32),
                pltpu.VMEM((8,), jnp.int32),
                pltpu.VMEM((8, 128), jnp.float32),
                pltpu.VMEM((8, D), jnp.float32),
                pltpu.VMEM((1, D), jnp.float32),
            ]
            + [pltpu.SemaphoreType.DMA] * 8
        ),
    )(gidx, eidx, cidx, bidx, gene_table, expr_table, cond_table, batch_table)


def kernel(gene, expr, cond, batch, pad, gene_table, bin_embeddings,
           W1, b1, W2, b2, cond_table, batch_table):
    expr_table = _expr_table(bin_embeddings, W1, b1, W2, b2)

    expr_table_rep = jnp.tile(expr_table, (NW, 1))

    zcol = jnp.zeros((C, 1), jnp.int32)
    gidx = jnp.concatenate([zcol, gene], axis=1).reshape(N // BLK, BLK)
    eidx = jnp.concatenate([zcol, expr], axis=1).reshape(-1)
    eidx = eidx + NUM_BINS * (jnp.arange(N, dtype=jnp.int32) // ROWS_PER_W)
    eidx = eidx.reshape(N // BLK, BLK)
    cidx = cond.reshape(-1)
    bidx = jnp.concatenate(
        [batch.reshape(NW, 2), jnp.zeros((NW, 6), jnp.int32)], axis=1
    ).reshape(-1)

    cond_table_p = jnp.pad(cond_table, ((0, 0), (0, 64)))

    out = _sc_assemble(gidx, eidx, cidx, bidx, gene_table, expr_table_rep,
                       cond_table_p, batch_table)

    final_emb = out.reshape(C, L, TWO_D)
    key_padding_mask = jnp.concatenate(
        [jnp.zeros((C, 1), dtype=bool), pad.astype(bool)], axis=1)
    return (final_emb, key_padding_mask)


# speedup vs baseline: 1.0130x; 1.0130x over previous
import functools

import jax
import jax.numpy as jnp
from jax import lax
from jax.experimental import pallas as pl
from jax.experimental.pallas import tpu as pltpu
from jax.experimental.pallas import tpu_sc as plsc

C, L1, D = 64, 2047, 256
NUM_BINS, HID = 32, 128
L = L1 + 1
N = C * L
TWO_D = 2 * D

NC, NS = 2, 16
NW = NC * NS
ROWS_PER_W = N // NW
BLK = 64
NCHUNK = ROWS_PER_W // BLK
RING = 2
LAG = 1


def _expr_table_body(bins_ref, w1_ref, b1_ref, w2_ref, b2_ref, out_ref):
    nb, d = out_ref.shape
    bins = bins_ref[...]
    vals = lax.broadcasted_iota(jnp.int32, (nb, 1), 0).astype(jnp.float32)
    h = jnp.maximum(vals * w1_ref[...] + b1_ref[...], 0.0)
    enc = lax.dot_general(h, w2_ref[...], (((1,), (0,)), ((), ())),
                          precision=lax.Precision.DEFAULT) + b2_ref[...]
    sim = lax.dot_general(enc, bins, (((1,), (1,)), ((), ())),
                          precision=lax.Precision.DEFAULT)
    col = lax.broadcasted_iota(jnp.int32, (nb, nb), 1)
    sim = jnp.where(col == 0, -1e30, sim)
    m = jnp.max(sim, axis=-1, keepdims=True)
    p = jnp.exp(sim - m)
    p = p / jnp.sum(p, axis=-1, keepdims=True)
    emb = lax.dot_general(p, bins, (((1,), (0,)), ((), ())),
                          precision=lax.Precision.DEFAULT)
    row = lax.broadcasted_iota(jnp.int32, (nb, d), 0)
    out_ref[...] = jnp.where(row == 0, bins[0:1, :], emb)


def _expr_table(bin_embeddings, w1, b1, w2, b2):
    return pl.pallas_call(
        _expr_table_body,
        out_shape=jax.ShapeDtypeStruct((NUM_BINS, D), jnp.float32),
    )(bin_embeddings, w1.reshape(1, HID), b1.reshape(1, HID),
      w2, b2.reshape(1, D))


def _sc_body(gidx_hbm, eidx_hbm, cidx_hbm, bidx_hbm,
             gene_t, expr_t, cond_t, batch_t, out_hbm, *scr):
    gidx_v, eidx_v = scr[0], scr[1]
    obuf = scr[2:2 + RING]
    ci_v, bi_v, cbuf, bbuf, fbuf = scr[2 + RING:7 + RING]
    sem_g = scr[7 + RING:7 + 2 * RING]
    sem_w = scr[7 + 2 * RING:7 + 3 * RING]

    wid = lax.axis_index("s") * NC + lax.axis_index("c")
    base_w = pl.multiple_of(wid * ROWS_PER_W, ROWS_PER_W)
    chunk0 = pl.multiple_of(wid * NCHUNK, NCHUNK)

    pltpu.sync_copy(gidx_hbm.at[pl.ds(chunk0, NCHUNK)], gidx_v)
    pltpu.sync_copy(eidx_hbm.at[pl.ds(chunk0, NCHUNK)], eidx_v)

    def issue_gathers(i, p):
        pltpu.async_copy(gene_t.at[gidx_v.at[i]], obuf[p].at[:, pl.ds(0, D)],
                         sem_g[p])
        pltpu.async_copy(expr_t.at[eidx_v.at[i]], obuf[p].at[:, pl.ds(D, D)],
                         sem_g[p])

    def wait_gathers(i, p):
        pltpu.make_async_copy(gene_t.at[gidx_v.at[i]],
                              obuf[p].at[:, pl.ds(0, D)], sem_g[p]).wait()
        pltpu.make_async_copy(expr_t.at[eidx_v.at[i]],
                              obuf[p].at[:, pl.ds(D, D)], sem_g[p]).wait()

    def issue_write(i, p):
        base = pl.multiple_of(base_w + i * BLK, BLK)
        pltpu.async_copy(obuf[p], out_hbm.at[pl.ds(base, BLK)], sem_w[p])

    def wait_write(p):
        pltpu.make_async_copy(obuf[p], out_hbm.at[pl.ds(base_w, BLK)],
                              sem_w[p]).wait()

    for i in range(LAG):
        issue_gathers(i, i)

    def step(i, p):
        pb = (p + LAG) % RING

        @pl.when(i + LAG < NCHUNK)
        def _():
            @pl.when(i + LAG >= RING)
            def _():
                wait_write(pb)
            issue_gathers(i + LAG, pb)

        wait_gathers(i, p)
        issue_write(i, p)

    def group(j, carry):
        for u in range(RING):
            step(RING * j + u, u)
        return carry

    lax.fori_loop(0, NCHUNK // RING, group, 0)
    for w in range(NCHUNK - RING + LAG, NCHUNK):
        wait_write(w % RING)

    c0 = 2 * wid
    pltpu.sync_copy(cidx_hbm.at[pl.ds(pl.multiple_of(c0 * 4, 8), 8)], ci_v)
    pltpu.sync_copy(bidx_hbm.at[pl.ds(pl.multiple_of(wid * 8, 8), 8)], bi_v)
    c_dma = pltpu.async_copy(cond_t.at[ci_v], cbuf, sem_g[0])
    b_dma = pltpu.async_copy(batch_t.at[bi_v], bbuf, sem_g[1 % RING])
    c_dma.wait()
    b_dma.wait()
    for cell in range(2):
        row = (c0 + cell) * L
        for j in range(D // 16):
            p = j * 16
            fbuf[0, pl.ds(p, 16)] = cbuf[4 * cell + p // 64, pl.ds(p % 64, 16)]
        pltpu.sync_copy(fbuf, out_hbm.at[pl.ds(row, 1), pl.ds(0, D)])
        pltpu.sync_copy(bbuf.at[cell], out_hbm.at[row, pl.ds(D, D)])


@functools.partial(jax.jit, static_argnames=())
def _sc_assemble(gidx, eidx, cidx, bidx, gene_table, expr_table,
                 cond_table, batch_table):
    mesh = plsc.VectorSubcoreMesh(core_axis_name="c", subcore_axis_name="s")
    return pl.kernel(
        _sc_body,
        out_type=jax.ShapeDtypeStruct((N, TWO_D), jnp.float32),
        mesh=mesh,
        scratch_types=(
            [pltpu.VMEM((NCHUNK, BLK), jnp.int32)] * 2
            + [pltpu.VMEM((BLK, TWO_D), jnp.float32)] * RING
            + [
                pltpu.VMEM((8,), jnp.int32),
                pltpu.VMEM((8,), jnp.int32),
                pltpu.VMEM((8, 128), jnp.float32),
                pltpu.VMEM((8, D), jnp.float32),
                pltpu.VMEM((1, D), jnp.float32),
            ]
            + [pltpu.SemaphoreType.DMA] * (2 * RING)
        ),
    )(gidx, eidx, cidx, bidx, gene_table, expr_table, cond_table, batch_table)


def kernel(gene, expr, cond, batch, pad, gene_table, bin_embeddings,
           W1, b1, W2, b2, cond_table, batch_table):
    expr_table = _expr_table(bin_embeddings, W1, b1, W2, b2)

    expr_table_rep = jnp.tile(expr_table, (NW, 1))

    zcol = jnp.zeros((C, 1), jnp.int32)
    gidx = jnp.concatenate([zcol, gene], axis=1).reshape(N // BLK, BLK)
    eidx = jnp.concatenate([zcol, expr], axis=1).reshape(-1)
    eidx = eidx + NUM_BINS * (jnp.arange(N, dtype=jnp.int32) // ROWS_PER_W)
    eidx = eidx.reshape(N // BLK, BLK)
    cidx = cond.reshape(-1)
    bidx = jnp.concatenate(
        [batch.reshape(NW, 2), jnp.zeros((NW, 6), jnp.int32)], axis=1
    ).reshape(-1)

    cond_table_p = jnp.pad(cond_table, ((0, 0), (0, 64)))

    out = _sc_assemble(gidx, eidx, cidx, bidx, gene_table, expr_table_rep,
                       cond_table_p, batch_table)

    final_emb = out.reshape(C, L, TWO_D)
    key_padding_mask = jnp.concatenate(
        [jnp.zeros((C, 1), dtype=bool), pad.astype(bool)], axis=1)
    return (final_emb, key_padding_mask)
